# Initial kernel scaffold; baseline (speedup 1.0000x reference)
import jax
import jax.numpy as jnp
from jax.experimental import pallas as pl


def kernel(pred, target, x, pos, edge_index, edge_attr):
    raise NotImplementedError("write your pallas kernel here")


# speedup vs baseline: 69.6063x; 69.6063x over previous
import functools

import jax
import jax.numpy as jnp
from jax import lax
from jax.experimental import pallas as pl
from jax.experimental.pallas import tpu as pltpu
from jax.experimental.pallas import tpu_sc as plsc

N_NODES = 10000
N_EDGES = 320000
EPS = 1e-8
ALPHA_DT = 0.6 / (1000.0 * 4186.0) * 1e-05

LANES = 16
NC = 2
NS = 16
NW = NC * NS
NPAD = 10240
EPW = N_EDGES // NW
SLICE = NPAD // NS

_mesh = plsc.VectorSubcoreMesh(
    core_axis_name="c", subcore_axis_name="s", num_cores=NC, num_subcores=NS
)
_sc_params = pltpu.CompilerParams(needs_layout_passes=False)


def _zero_refs(refs):
    z = jnp.zeros((LANES,), jnp.float32)

    def body(i, _):
        for r in refs:
            r[pl.ds(i * LANES, LANES)] = z
        return 0

    lax.fori_loop(0, NPAD // LANES, body, 0)


def _reduce_rows(red, redout):
    def body(j, _):
        o = j * LANES
        acc = red[pl.ds(o, LANES)]
        for r in range(1, NS):
            acc = acc + red[pl.ds(r * SLICE + o, LANES)]
        redout[pl.ds(o, LANES)] = acc
        return 0

    lax.fori_loop(0, SLICE // LANES, body, 0)


@functools.partial(
    pl.kernel,
    out_type=[jax.ShapeDtypeStruct((NC * NPAD,), jnp.float32)] * 4,
    mesh=_mesh,
    scratch_types=[
        pltpu.VMEM((NPAD,), jnp.float32),
        pltpu.VMEM((NPAD,), jnp.float32),
        pltpu.VMEM((NPAD,), jnp.float32),
        pltpu.VMEM((NPAD,), jnp.float32),
        pltpu.VMEM((NPAD,), jnp.float32),
        pltpu.VMEM((NPAD,), jnp.float32),
        pltpu.VMEM((NPAD,), jnp.float32),
        pltpu.VMEM((NPAD,), jnp.float32),
        pltpu.VMEM((EPW,), jnp.int32),
        pltpu.VMEM((EPW,), jnp.int32),
        pltpu.VMEM((NS * SLICE,), jnp.float32),
        pltpu.VMEM((SLICE,), jnp.float32),
        pltpu.VMEM_SHARED((NS * NPAD,), jnp.float32),
    ],
    compiler_params=_sc_params,
)
def _grad_kernel(T_h, px_h, py_h, pz_h, src_h, dst_h,
                 onx, ony, onz, ocnt,
                 tT, tpx, tpy, tpz, ax, ay, az, ac,
                 esrc, edst, red, redout, sh):
    cid = lax.axis_index("c")
    sid = lax.axis_index("s")
    wid = cid * NS + sid
    ebase = wid * EPW

    pltpu.sync_copy(T_h, tT)
    pltpu.sync_copy(px_h, tpx)
    pltpu.sync_copy(py_h, tpy)
    pltpu.sync_copy(pz_h, tpz)
    pltpu.sync_copy(src_h.at[pl.ds(ebase, EPW)], esrc)
    pltpu.sync_copy(dst_h.at[pl.ds(ebase, EPW)], edst)

    _zero_refs([ax, ay, az, ac])
    ones = jnp.full((LANES,), 1.0, jnp.float32)

    def edge_body(i, _):
        o = pl.ds(i * LANES, LANES)
        s = esrc[o]
        d = edst[o]
        ts = plsc.load_gather(tT, [s])
        td = plsc.load_gather(tT, [d])
        pxs = plsc.load_gather(tpx, [s])
        pxd = plsc.load_gather(tpx, [d])
        pys = plsc.load_gather(tpy, [s])
        pyd = plsc.load_gather(tpy, [d])
        pzs = plsc.load_gather(tpz, [s])
        pzd = plsc.load_gather(tpz, [d])
        dx = pxd - pxs
        dy = pyd - pys
        dz = pzd - pzs
        dist2 = dx * dx + dy * dy + dz * dz + EPS
        w = (td - ts) / dist2
        plsc.addupdate_scatter(ax, [d], w * dx)
        plsc.addupdate_scatter(ay, [d], w * dy)
        plsc.addupdate_scatter(az, [d], w * dz)
        plsc.addupdate_scatter(ac, [d], ones)
        return 0

    lax.fori_loop(0, EPW // LANES, edge_body, 0)

    nbase = sid * SLICE
    for k, (acc, out) in enumerate([(ax, onx), (ay, ony), (az, onz), (ac, ocnt)]):
        if k > 0:
            plsc.subcore_barrier()
        pltpu.sync_copy(acc, sh.at[pl.ds(sid * NPAD, NPAD)])
        plsc.subcore_barrier()
        for r in range(NS):
            pltpu.sync_copy(sh.at[pl.ds(r * NPAD + nbase, SLICE)],
                            red.at[pl.ds(r * SLICE, SLICE)])
        _reduce_rows(red, redout)
        pltpu.sync_copy(redout, out.at[pl.ds(cid * NPAD + nbase, SLICE)])


@functools.partial(
    pl.kernel,
    out_type=jax.ShapeDtypeStruct((NC * NPAD,), jnp.float32),
    mesh=_mesh,
    scratch_types=[
        pltpu.VMEM((NPAD,), jnp.float32),
        pltpu.VMEM((NPAD,), jnp.float32),
        pltpu.VMEM((NPAD,), jnp.float32),
        pltpu.VMEM((NPAD,), jnp.float32),
        pltpu.VMEM((NPAD,), jnp.float32),
        pltpu.VMEM((NPAD,), jnp.float32),
        pltpu.VMEM((NPAD,), jnp.float32),
        pltpu.VMEM((EPW,), jnp.int32),
        pltpu.VMEM((EPW,), jnp.int32),
        pltpu.VMEM((NS * SLICE,), jnp.float32),
        pltpu.VMEM((SLICE,), jnp.float32),
        pltpu.VMEM((NC * SLICE,), jnp.float32),
        pltpu.VMEM((SLICE,), jnp.float32),
        pltpu.VMEM_SHARED((3 * NPAD,), jnp.float32),
        pltpu.VMEM_SHARED((NS * NPAD,), jnp.float32),
    ],
    compiler_params=_sc_params,
)
def _div_kernel(pnx, pny, pnz, pcnt, px_h, py_h, pz_h, src_h, dst_h,
                odiv,
                tpx, tpy, tpz, tgx, tgy, tgz, adiv,
                esrc, edst, red, redout, pbuf, ccnt, shg, sh):
    cid = lax.axis_index("c")
    sid = lax.axis_index("s")
    wid = cid * NS + sid
    ebase = wid * EPW
    nbase = sid * SLICE

    pltpu.sync_copy(px_h, tpx)
    pltpu.sync_copy(py_h, tpy)
    pltpu.sync_copy(pz_h, tpz)
    pltpu.sync_copy(src_h.at[pl.ds(ebase, EPW)], esrc)
    pltpu.sync_copy(dst_h.at[pl.ds(ebase, EPW)], edst)

    pltpu.sync_copy(pcnt.at[pl.ds(nbase, SLICE)], pbuf.at[pl.ds(0, SLICE)])
    pltpu.sync_copy(pcnt.at[pl.ds(NPAD + nbase, SLICE)],
                    pbuf.at[pl.ds(SLICE, SLICE)])

    def cnt_body(j, _):
        o = j * LANES
        ccnt[pl.ds(o, LANES)] = (
            pbuf[pl.ds(o, LANES)] + pbuf[pl.ds(SLICE + o, LANES)] + EPS
        )
        return 0

    lax.fori_loop(0, SLICE // LANES, cnt_body, 0)

    for k, pn in enumerate([pnx, pny, pnz]):
        pltpu.sync_copy(pn.at[pl.ds(nbase, SLICE)], pbuf.at[pl.ds(0, SLICE)])
        pltpu.sync_copy(pn.at[pl.ds(NPAD + nbase, SLICE)],
                        pbuf.at[pl.ds(SLICE, SLICE)])

        def g_body(j, _):
            o = j * LANES
            redout[pl.ds(o, LANES)] = (
                pbuf[pl.ds(o, LANES)] + pbuf[pl.ds(SLICE + o, LANES)]
            ) / ccnt[pl.ds(o, LANES)]
            return 0

        lax.fori_loop(0, SLICE // LANES, g_body, 0)
        pltpu.sync_copy(redout, shg.at[pl.ds(k * NPAD + nbase, SLICE)])

    plsc.subcore_barrier()
    pltpu.sync_copy(shg.at[pl.ds(0, NPAD)], tgx)
    pltpu.sync_copy(shg.at[pl.ds(NPAD, NPAD)], tgy)
    pltpu.sync_copy(shg.at[pl.ds(2 * NPAD, NPAD)], tgz)

    _zero_refs([adiv])

    def edge_body(i, _):
        o = pl.ds(i * LANES, LANES)
        s = esrc[o]
        d = edst[o]
        pxs = plsc.load_gather(tpx, [s])
        pxd = plsc.load_gather(tpx, [d])
        pys = plsc.load_gather(tpy, [s])
        pyd = plsc.load_gather(tpy, [d])
        pzs = plsc.load_gather(tpz, [s])
        pzd = plsc.load_gather(tpz, [d])
        gxs = plsc.load_gather(tgx, [s])
        gxd = plsc.load_gather(tgx, [d])
        gys = plsc.load_gather(tgy, [s])
        gyd = plsc.load_gather(tgy, [d])
        gzs = plsc.load_gather(tgz, [s])
        gzd = plsc.load_gather(tgz, [d])
        dx = pxd - pxs
        dy = pyd - pys
        dz = pzd - pzs
        dist2 = dx * dx + dy * dy + dz * dz + EPS
        dive = ((gxd - gxs) * dx + (gyd - gys) * dy + (gzd - gzs) * dz) / dist2
        plsc.addupdate_scatter(adiv, [d], dive)
        return 0

    lax.fori_loop(0, EPW // LANES, edge_body, 0)

    pltpu.sync_copy(adiv, sh.at[pl.ds(sid * NPAD, NPAD)])
    plsc.subcore_barrier()
    for r in range(NS):
        pltpu.sync_copy(sh.at[pl.ds(r * NPAD + nbase, SLICE)],
                        red.at[pl.ds(r * SLICE, SLICE)])
    _reduce_rows(red, redout)
    pltpu.sync_copy(redout, odiv.at[pl.ds(cid * NPAD + nbase, SLICE)])


def _loss_body(div_ref, cnt_ref, dtp_ref, out_ref):
    d = div_ref[0:1, :] + div_ref[1:2, :]
    c = cnt_ref[0:1, :] + cnt_ref[1:2, :]
    lap = d / (c + EPS)
    diff = dtp_ref[...] - ALPHA_DT * lap
    out_ref[...] = jnp.sum(diff * diff, keepdims=True) * (1.0 / N_NODES)


_loss_call = pl.pallas_call(
    _loss_body,
    out_shape=jax.ShapeDtypeStruct((1, 1), jnp.float32),
)


def kernel(pred, target, x, pos, edge_index, edge_attr):
    padn = NPAD - N_NODES
    T = jnp.pad(x[:, 3], (0, padn))
    dtp = jnp.pad(pred[:, 0], (0, padn)).reshape(1, NPAD)
    px = jnp.pad(pos[:, 0], (0, padn))
    py = jnp.pad(pos[:, 1], (0, padn))
    pz = jnp.pad(pos[:, 2], (0, padn))
    ei = edge_index.astype(jnp.int32)
    src = ei[0]
    dst = ei[1]

    nx, ny, nz, cnt = _grad_kernel(T, px, py, pz, src, dst)
    divp = _div_kernel(nx, ny, nz, cnt, px, py, pz, src, dst)
    loss = _loss_call(divp.reshape(NC, NPAD), cnt.reshape(NC, NPAD), dtp)
    return loss[0, 0]


# speedup vs baseline: 88.3291x; 1.2690x over previous
import functools

import jax
import jax.numpy as jnp
from jax import lax
from jax.experimental import pallas as pl
from jax.experimental.pallas import tpu as pltpu
from jax.experimental.pallas import tpu_sc as plsc

N_NODES = 10000
N_EDGES = 320000
EPS = 1e-8
ALPHA_DT = 0.6 / (1000.0 * 4186.0) * 1e-05

LANES = 16
NC = 2
NS = 16
NW = NC * NS
NPAD = 10240
EPW = N_EDGES // NW
SLICE = NPAD // NS

_mesh = plsc.VectorSubcoreMesh(
    core_axis_name="c", subcore_axis_name="s", num_cores=NC, num_subcores=NS
)
_sc_params = pltpu.CompilerParams(needs_layout_passes=False)


def _zero_refs(refs):
    z = jnp.zeros((LANES,), jnp.float32)

    @plsc.parallel_loop(0, NPAD // LANES, unroll=4)
    def _(i):
        for r in refs:
            r[pl.ds(i * LANES, LANES)] = z


def _reduce_rows(red, redout):
    @plsc.parallel_loop(0, SLICE // LANES, unroll=2)
    def _(j):
        o = j * LANES
        acc = red[pl.ds(o, LANES)]
        for r in range(1, NS):
            acc = acc + red[pl.ds(r * SLICE + o, LANES)]
        redout[pl.ds(o, LANES)] = acc


@functools.partial(
    pl.kernel,
    out_type=[jax.ShapeDtypeStruct((NC * NPAD,), jnp.float32)] * 4,
    mesh=_mesh,
    scratch_types=[
        pltpu.VMEM((NPAD,), jnp.float32),
        pltpu.VMEM((NPAD,), jnp.float32),
        pltpu.VMEM((NPAD,), jnp.float32),
        pltpu.VMEM((NPAD,), jnp.float32),
        pltpu.VMEM((NPAD,), jnp.float32),
        pltpu.VMEM((NPAD,), jnp.float32),
        pltpu.VMEM((NPAD,), jnp.float32),
        pltpu.VMEM((NPAD,), jnp.float32),
        pltpu.VMEM((EPW,), jnp.int32),
        pltpu.VMEM((EPW,), jnp.int32),
        pltpu.VMEM((NS * SLICE,), jnp.float32),
        pltpu.VMEM((SLICE,), jnp.float32),
        pltpu.VMEM_SHARED((NS * NPAD,), jnp.float32),
    ],
    compiler_params=_sc_params,
)
def _grad_kernel(T_h, px_h, py_h, pz_h, src_h, dst_h,
                 onx, ony, onz, ocnt,
                 tT, tpx, tpy, tpz, ax, ay, az, ac,
                 esrc, edst, red, redout, sh):
    cid = lax.axis_index("c")
    sid = lax.axis_index("s")
    wid = cid * NS + sid
    ebase = wid * EPW

    pltpu.sync_copy(T_h, tT)
    pltpu.sync_copy(px_h, tpx)
    pltpu.sync_copy(py_h, tpy)
    pltpu.sync_copy(pz_h, tpz)
    pltpu.sync_copy(src_h.at[pl.ds(ebase, EPW)], esrc)
    pltpu.sync_copy(dst_h.at[pl.ds(ebase, EPW)], edst)

    _zero_refs([ax, ay, az, ac])
    ones = jnp.full((LANES,), 1.0, jnp.float32)

    @plsc.parallel_loop(0, EPW // LANES, unroll=4)
    def _(i):
        o = pl.ds(i * LANES, LANES)
        s = esrc[o]
        d = edst[o]
        ts = plsc.load_gather(tT, [s])
        td = plsc.load_gather(tT, [d])
        pxs = plsc.load_gather(tpx, [s])
        pxd = plsc.load_gather(tpx, [d])
        pys = plsc.load_gather(tpy, [s])
        pyd = plsc.load_gather(tpy, [d])
        pzs = plsc.load_gather(tpz, [s])
        pzd = plsc.load_gather(tpz, [d])
        dx = pxd - pxs
        dy = pyd - pys
        dz = pzd - pzs
        dist2 = dx * dx + dy * dy + dz * dz + EPS
        w = (td - ts) / dist2
        plsc.addupdate_scatter(ax, [d], w * dx)
        plsc.addupdate_scatter(ay, [d], w * dy)
        plsc.addupdate_scatter(az, [d], w * dz)
        plsc.addupdate_scatter(ac, [d], ones)

    nbase = sid * SLICE
    for k, (acc, out) in enumerate([(ax, onx), (ay, ony), (az, onz), (ac, ocnt)]):
        if k > 0:
            plsc.subcore_barrier()
        pltpu.sync_copy(acc, sh.at[pl.ds(sid * NPAD, NPAD)])
        plsc.subcore_barrier()
        for r in range(NS):
            pltpu.sync_copy(sh.at[pl.ds(r * NPAD + nbase, SLICE)],
                            red.at[pl.ds(r * SLICE, SLICE)])
        _reduce_rows(red, redout)
        pltpu.sync_copy(redout, out.at[pl.ds(cid * NPAD + nbase, SLICE)])


@functools.partial(
    pl.kernel,
    out_type=jax.ShapeDtypeStruct((NC * NPAD,), jnp.float32),
    mesh=_mesh,
    scratch_types=[
        pltpu.VMEM((NPAD,), jnp.float32),
        pltpu.VMEM((NPAD,), jnp.float32),
        pltpu.VMEM((NPAD,), jnp.float32),
        pltpu.VMEM((NPAD,), jnp.float32),
        pltpu.VMEM((NPAD,), jnp.float32),
        pltpu.VMEM((NPAD,), jnp.float32),
        pltpu.VMEM((NPAD,), jnp.float32),
        pltpu.VMEM((EPW,), jnp.int32),
        pltpu.VMEM((EPW,), jnp.int32),
        pltpu.VMEM((NS * SLICE,), jnp.float32),
        pltpu.VMEM((SLICE,), jnp.float32),
        pltpu.VMEM((NC * SLICE,), jnp.float32),
        pltpu.VMEM((SLICE,), jnp.float32),
        pltpu.VMEM_SHARED((3 * NPAD,), jnp.float32),
        pltpu.VMEM_SHARED((NS * NPAD,), jnp.float32),
    ],
    compiler_params=_sc_params,
)
def _div_kernel(pnx, pny, pnz, pcnt, px_h, py_h, pz_h, src_h, dst_h,
                odiv,
                tpx, tpy, tpz, tgx, tgy, tgz, adiv,
                esrc, edst, red, redout, pbuf, ccnt, shg, sh):
    cid = lax.axis_index("c")
    sid = lax.axis_index("s")
    wid = cid * NS + sid
    ebase = wid * EPW
    nbase = sid * SLICE

    pltpu.sync_copy(px_h, tpx)
    pltpu.sync_copy(py_h, tpy)
    pltpu.sync_copy(pz_h, tpz)
    pltpu.sync_copy(src_h.at[pl.ds(ebase, EPW)], esrc)
    pltpu.sync_copy(dst_h.at[pl.ds(ebase, EPW)], edst)

    pltpu.sync_copy(pcnt.at[pl.ds(nbase, SLICE)], pbuf.at[pl.ds(0, SLICE)])
    pltpu.sync_copy(pcnt.at[pl.ds(NPAD + nbase, SLICE)],
                    pbuf.at[pl.ds(SLICE, SLICE)])

    def cnt_body(j, _):
        o = j * LANES
        ccnt[pl.ds(o, LANES)] = (
            pbuf[pl.ds(o, LANES)] + pbuf[pl.ds(SLICE + o, LANES)] + EPS
        )
        return 0

    lax.fori_loop(0, SLICE // LANES, cnt_body, 0)

    for k, pn in enumerate([pnx, pny, pnz]):
        pltpu.sync_copy(pn.at[pl.ds(nbase, SLICE)], pbuf.at[pl.ds(0, SLICE)])
        pltpu.sync_copy(pn.at[pl.ds(NPAD + nbase, SLICE)],
                        pbuf.at[pl.ds(SLICE, SLICE)])

        def g_body(j, _):
            o = j * LANES
            redout[pl.ds(o, LANES)] = (
                pbuf[pl.ds(o, LANES)] + pbuf[pl.ds(SLICE + o, LANES)]
            ) / ccnt[pl.ds(o, LANES)]
            return 0

        lax.fori_loop(0, SLICE // LANES, g_body, 0)
        pltpu.sync_copy(redout, shg.at[pl.ds(k * NPAD + nbase, SLICE)])

    plsc.subcore_barrier()
    pltpu.sync_copy(shg.at[pl.ds(0, NPAD)], tgx)
    pltpu.sync_copy(shg.at[pl.ds(NPAD, NPAD)], tgy)
    pltpu.sync_copy(shg.at[pl.ds(2 * NPAD, NPAD)], tgz)

    _zero_refs([adiv])

    @plsc.parallel_loop(0, EPW // LANES, unroll=4)
    def _(i):
        o = pl.ds(i * LANES, LANES)
        s = esrc[o]
        d = edst[o]
        pxs = plsc.load_gather(tpx, [s])
        pxd = plsc.load_gather(tpx, [d])
        pys = plsc.load_gather(tpy, [s])
        pyd = plsc.load_gather(tpy, [d])
        pzs = plsc.load_gather(tpz, [s])
        pzd = plsc.load_gather(tpz, [d])
        gxs = plsc.load_gather(tgx, [s])
        gxd = plsc.load_gather(tgx, [d])
        gys = plsc.load_gather(tgy, [s])
        gyd = plsc.load_gather(tgy, [d])
        gzs = plsc.load_gather(tgz, [s])
        gzd = plsc.load_gather(tgz, [d])
        dx = pxd - pxs
        dy = pyd - pys
        dz = pzd - pzs
        dist2 = dx * dx + dy * dy + dz * dz + EPS
        dive = ((gxd - gxs) * dx + (gyd - gys) * dy + (gzd - gzs) * dz) / dist2
        plsc.addupdate_scatter(adiv, [d], dive)

    pltpu.sync_copy(adiv, sh.at[pl.ds(sid * NPAD, NPAD)])
    plsc.subcore_barrier()
    for r in range(NS):
        pltpu.sync_copy(sh.at[pl.ds(r * NPAD + nbase, SLICE)],
                        red.at[pl.ds(r * SLICE, SLICE)])
    _reduce_rows(red, redout)
    pltpu.sync_copy(redout, odiv.at[pl.ds(cid * NPAD + nbase, SLICE)])


def _loss_body(div_ref, cnt_ref, dtp_ref, out_ref):
    d = div_ref[0:1, :] + div_ref[1:2, :]
    c = cnt_ref[0:1, :] + cnt_ref[1:2, :]
    lap = d / (c + EPS)
    diff = dtp_ref[...] - ALPHA_DT * lap
    out_ref[...] = jnp.sum(diff * diff, keepdims=True) * (1.0 / N_NODES)


_loss_call = pl.pallas_call(
    _loss_body,
    out_shape=jax.ShapeDtypeStruct((1, 1), jnp.float32),
)


def kernel(pred, target, x, pos, edge_index, edge_attr):
    padn = NPAD - N_NODES
    T = jnp.pad(x[:, 3], (0, padn))
    dtp = jnp.pad(pred[:, 0], (0, padn)).reshape(1, NPAD)
    px = jnp.pad(pos[:, 0], (0, padn))
    py = jnp.pad(pos[:, 1], (0, padn))
    pz = jnp.pad(pos[:, 2], (0, padn))
    ei = edge_index.astype(jnp.int32)
    src = ei[0]
    dst = ei[1]

    nx, ny, nz, cnt = _grad_kernel(T, px, py, pz, src, dst)
    divp = _div_kernel(nx, ny, nz, cnt, px, py, pz, src, dst)
    loss = _loss_call(divp.reshape(NC, NPAD), cnt.reshape(NC, NPAD), dtp)
    return loss[0, 0]
